# baseline (speedup 1.0000x reference)
import functools

import jax
import jax.numpy as jnp
from jax import lax
from jax.experimental import pallas as pl
from jax.experimental.pallas import tpu as pltpu
from jax.experimental.pallas import tpu_sc as plsc

VOCAB = 100000
HIDDEN = 64
NUM_CLASSES = 3
BATCH = 4096
SEQ = 200

PAD = 16
NC, NS = 2, 16
NW = NC * NS
VPAD = 100352
BPW = BATCH // NW
CB = 8
NCHUNK = BPW // CB
ROWS = CB * SEQ


def _proj_body(wt_ref, b_ref, embt_ref, out_ref):
    x = lax.dot_general(
        wt_ref[...], embt_ref[...],
        (((0,), (0,)), ((), ())),
        preferred_element_type=jnp.float32,
    )
    out_ref[...] = x + b_ref[...]


def _project(emb_t, wt_pad, bcol):
    blk = 8192
    return pl.pallas_call(
        _proj_body,
        grid=((VPAD + blk - 1) // blk,),
        in_specs=[pl.BlockSpec((HIDDEN, PAD), lambda i: (0, 0)),
                  pl.BlockSpec((PAD, 1), lambda i: (0, 0)),
                  pl.BlockSpec((HIDDEN, blk), lambda i: (0, i))],
        out_specs=pl.BlockSpec((PAD, blk), lambda i: (0, i)),
        out_shape=jax.ShapeDtypeStruct((PAD, VPAD), jnp.float32),
    )(wt_pad, bcol, emb_t)


def _sc_transpose(pt):
    mesh = plsc.VectorSubcoreMesh(core_axis_name="c", subcore_axis_name="s")
    CPW = VPAD // NW
    NG = CPW // 16

    HALF = CPW // 2
    NGH = HALF // 16

    @functools.partial(
        pl.kernel,
        mesh=mesh,
        out_type=jax.ShapeDtypeStruct((VPAD, PAD), jnp.float32),
        scratch_types=[
            pltpu.VMEM((PAD, HALF), jnp.float32),
            pltpu.VMEM((PAD, HALF), jnp.float32),
            pltpu.VMEM((HALF, PAD), jnp.float32),
            pltpu.VMEM((HALF, PAD), jnp.float32),
            pltpu.SemaphoreType.DMA,
            pltpu.SemaphoreType.DMA,
            pltpu.SemaphoreType.DMA,
            pltpu.SemaphoreType.DMA,
        ],
        compiler_params=pltpu.CompilerParams(use_tc_tiling_on_sc=False,
                                             needs_layout_passes=False),
    )
    def k(pt_hbm, out_hbm, slab0, slab1, out0, out1, si0, si1, so0, so1):
        wid = lax.axis_index("s") * NC + lax.axis_index("c")
        base = wid * CPW
        lanes = lax.iota(jnp.int32, 16)
        slab = (slab0, slab1)
        outv = (out0, out1)

        cpi = [pltpu.async_copy(pt_hbm.at[:, pl.ds(base, HALF)], slab0, si0),
               pltpu.async_copy(pt_hbm.at[:, pl.ds(base + HALF, HALF)],
                                slab1, si1)]
        cpo = [None, None]
        for h in range(2):
            cpi[h].wait()

            def grp(g, carry, _h=h):
                col0 = g * 16
                for i in range(16):
                    col = jnp.full((16,), col0 + i, jnp.int32)
                    row = plsc.load_gather(slab[_h], [lanes, col])
                    outv[_h][col0 + i] = row
                return carry

            lax.fori_loop(0, NGH, grp, 0, unroll=7)
            cpo[h] = pltpu.async_copy(
                outv[h], out_hbm.at[pl.ds(base + h * HALF, HALF)],
                so0 if h == 0 else so1)
        cpo[0].wait()
        cpo[1].wait()

    return k(pt)


def _sc_pool(ids_flat, ptab):
    mesh = plsc.VectorSubcoreMesh(core_axis_name="c", subcore_axis_name="s")

    @functools.partial(
        pl.kernel,
        mesh=mesh,
        out_type=jax.ShapeDtypeStruct((BATCH, PAD), jnp.float32),
        scratch_types=[
            pltpu.VMEM((BPW * SEQ,), jnp.int32),
            pltpu.VMEM((ROWS, PAD), jnp.float32),
            pltpu.VMEM((ROWS, PAD), jnp.float32),
            pltpu.VMEM((BPW, PAD), jnp.float32),
            pltpu.SemaphoreType.DMA,
            pltpu.SemaphoreType.DMA,
        ],
        compiler_params=pltpu.CompilerParams(use_tc_tiling_on_sc=False),
    )
    def k(ids_hbm, ptab_hbm, out_hbm,
          idx_v, rows0, rows1, out_v, sem0, sem1):
        wid = lax.axis_index("s") * NC + lax.axis_index("c")
        base = wid * BPW
        pltpu.sync_copy(ids_hbm.at[pl.ds(base * SEQ, BPW * SEQ)], idx_v)
        inv = jnp.float32(1.0 / SEQ)

        rows = (rows0, rows1)
        sem = (sem0, sem1)

        def start(c, p):
            return pltpu.async_copy(
                ptab_hbm.at[idx_v.at[pl.ds(c * ROWS, ROWS)]], rows[p],
                sem[p])

        def compute(c, p):
            rv = rows[p]

            def row_body(bi, carry):
                r0 = bi * SEQ

                def acc_body(j, acc):
                    o = r0 + j * 8
                    s = ((rv[o] + rv[o + 1]) + (rv[o + 2] + rv[o + 3])) \
                        + ((rv[o + 4] + rv[o + 5]) + (rv[o + 6] + rv[o + 7]))
                    return acc + s

                acc = lax.fori_loop(0, SEQ // 8, acc_body,
                                    jnp.zeros((PAD,), jnp.float32))
                out_v[c * CB + bi] = acc * inv
                return carry

            lax.fori_loop(0, CB, row_body, 0)

        cps = [start(0, 0), start(1, 1)]
        for c in range(NCHUNK):
            p = c % 2
            cps[p].wait()
            compute(c, p)
            if c + 2 < NCHUNK:
                cps[p] = start(c + 2, p)

        pltpu.sync_copy(out_v, out_hbm.at[pl.ds(base, BPW)])

    return k(ids_flat, ptab)


def kernel(input_ids, emb_table, W, b):
    wt_pad = jnp.pad(W.T, ((0, 0), (0, PAD - NUM_CLASSES)))
    bcol = jnp.pad(b, (0, PAD - NUM_CLASSES)).reshape(PAD, 1)
    ptab = _sc_transpose(_project(emb_table.T, wt_pad, bcol))
    ids_flat = input_ids.reshape(-1).astype(jnp.int32)
    out = _sc_pool(ids_flat, ptab)
    return out[:, :NUM_CLASSES]


# speedup vs baseline: 1.3112x; 1.3112x over previous
import functools

import jax
import jax.numpy as jnp
from jax import lax
from jax.experimental import pallas as pl
from jax.experimental.pallas import tpu as pltpu
from jax.experimental.pallas import tpu_sc as plsc

VOCAB = 100000
HIDDEN = 64
NUM_CLASSES = 3
BATCH = 4096
SEQ = 200

PAD = 16
NC, NS = 2, 16
NW = NC * NS
VPAD = 100352
BPW = BATCH // NW
CB = 8
NCHUNK = BPW // CB
ROWS = CB * SEQ


def _proj_body(w_ref, b_ref, embt_ref, out_ref):
    x = lax.dot_general(
        w_ref[...], embt_ref[...],
        (((1,), (0,)), ((), ())),
        preferred_element_type=jnp.float32,
    )
    x = x + b_ref[...]
    out_ref[...] = lax.pad(x, jnp.float32(0.0),
                           ((0, PAD - NUM_CLASSES, 0), (0, 0, 0)))


def _project(emb_t, w, bcol):
    blk = 8192
    return pl.pallas_call(
        _proj_body,
        grid=((VPAD + blk - 1) // blk,),
        in_specs=[pl.BlockSpec((NUM_CLASSES, HIDDEN), lambda i: (0, 0)),
                  pl.BlockSpec((NUM_CLASSES, 1), lambda i: (0, 0)),
                  pl.BlockSpec((HIDDEN, blk), lambda i: (0, i))],
        out_specs=pl.BlockSpec((PAD, blk), lambda i: (0, i)),
        out_shape=jax.ShapeDtypeStruct((PAD, VPAD), jnp.float32),
    )(w, bcol, emb_t)


def _sc_transpose(pt):
    mesh = plsc.VectorSubcoreMesh(core_axis_name="c", subcore_axis_name="s")
    CPW = VPAD // NW
    NG = CPW // 16

    HALF = CPW // 2
    NGH = HALF // 16

    @functools.partial(
        pl.kernel,
        mesh=mesh,
        out_type=jax.ShapeDtypeStruct((VPAD, PAD), jnp.float32),
        scratch_types=[
            pltpu.VMEM((PAD, HALF), jnp.float32),
            pltpu.VMEM((PAD, HALF), jnp.float32),
            pltpu.VMEM((HALF, PAD), jnp.float32),
            pltpu.VMEM((HALF, PAD), jnp.float32),
            pltpu.SemaphoreType.DMA,
            pltpu.SemaphoreType.DMA,
            pltpu.SemaphoreType.DMA,
            pltpu.SemaphoreType.DMA,
        ],
        compiler_params=pltpu.CompilerParams(use_tc_tiling_on_sc=False,
                                             needs_layout_passes=False),
    )
    def k(pt_hbm, out_hbm, slab0, slab1, out0, out1, si0, si1, so0, so1):
        wid = lax.axis_index("s") * NC + lax.axis_index("c")
        base = wid * CPW
        lanes = lax.iota(jnp.int32, 16)
        slab = (slab0, slab1)
        outv = (out0, out1)

        cpi = [pltpu.async_copy(pt_hbm.at[:, pl.ds(base, HALF)], slab0, si0),
               pltpu.async_copy(pt_hbm.at[:, pl.ds(base + HALF, HALF)],
                                slab1, si1)]
        cpo = [None, None]
        for h in range(2):
            cpi[h].wait()

            def grp(g, carry, _h=h):
                col0 = g * 16
                for r in range(PAD):
                    v = slab[_h][r, pl.ds(col0, 16)]
                    plsc.store_scatter(
                        outv[_h],
                        [col0 + lanes, jnp.full((16,), r, jnp.int32)], v)
                return carry

            lax.fori_loop(0, NGH, grp, 0)
            cpo[h] = pltpu.async_copy(
                outv[h], out_hbm.at[pl.ds(base + h * HALF, HALF)],
                so0 if h == 0 else so1)
        cpo[0].wait()
        cpo[1].wait()

    return k(pt)


def _sc_pool(ids_flat, ptab):
    mesh = plsc.VectorSubcoreMesh(core_axis_name="c", subcore_axis_name="s")

    @functools.partial(
        pl.kernel,
        mesh=mesh,
        out_type=jax.ShapeDtypeStruct((BATCH, PAD), jnp.float32),
        scratch_types=[
            pltpu.VMEM((BPW * SEQ,), jnp.int32),
            pltpu.VMEM((ROWS, PAD), jnp.float32),
            pltpu.VMEM((ROWS, PAD), jnp.float32),
            pltpu.VMEM((BPW, PAD), jnp.float32),
            pltpu.SemaphoreType.DMA,
            pltpu.SemaphoreType.DMA,
        ],
        compiler_params=pltpu.CompilerParams(use_tc_tiling_on_sc=False),
    )
    def k(ids_hbm, ptab_hbm, out_hbm,
          idx_v, rows0, rows1, out_v, sem0, sem1):
        wid = lax.axis_index("s") * NC + lax.axis_index("c")
        base = wid * BPW
        pltpu.sync_copy(ids_hbm.at[pl.ds(base * SEQ, BPW * SEQ)], idx_v)
        inv = jnp.float32(1.0 / SEQ)

        rows = (rows0, rows1)
        sem = (sem0, sem1)

        def start(c, p):
            return pltpu.async_copy(
                ptab_hbm.at[idx_v.at[pl.ds(c * ROWS, ROWS)]], rows[p],
                sem[p])

        def compute(c, p):
            rv = rows[p]

            def row_body(bi, carry):
                r0 = bi * SEQ

                def acc_body(j, acc):
                    o = r0 + j * 8
                    s = ((rv[o] + rv[o + 1]) + (rv[o + 2] + rv[o + 3])) \
                        + ((rv[o + 4] + rv[o + 5]) + (rv[o + 6] + rv[o + 7]))
                    return acc + s

                acc = lax.fori_loop(0, SEQ // 8, acc_body,
                                    jnp.zeros((PAD,), jnp.float32))
                out_v[c * CB + bi] = acc * inv
                return carry

            lax.fori_loop(0, CB, row_body, 0)

        cps = [start(0, 0), start(1, 1)]
        for c in range(NCHUNK):
            p = c % 2
            cps[p].wait()
            compute(c, p)
            if c + 2 < NCHUNK:
                cps[p] = start(c + 2, p)

        pltpu.sync_copy(out_v, out_hbm.at[pl.ds(base, BPW)])

    return k(ids_flat, ptab)


def kernel(input_ids, emb_table, W, b):
    bcol = b.reshape(NUM_CLASSES, 1)
    ptab = _sc_transpose(_project(emb_table.T, W, bcol))
    ids_flat = input_ids.reshape(-1).astype(jnp.int32)
    out = _sc_pool(ids_flat, ptab)
    return out[:, :NUM_CLASSES]


# speedup vs baseline: 1.3517x; 1.0309x over previous
import functools

import jax
import jax.numpy as jnp
from jax import lax
from jax.experimental import pallas as pl
from jax.experimental.pallas import tpu as pltpu
from jax.experimental.pallas import tpu_sc as plsc

VOCAB = 100000
HIDDEN = 64
NUM_CLASSES = 3
BATCH = 4096
SEQ = 200

PAD = 16
NC, NS = 2, 16
NW = NC * NS
VPAD = 100352
BPW = BATCH // NW
CB = 8
NCHUNK = BPW // CB
ROWS = CB * SEQ


def _proj_body(w_ref, b_ref, embt_ref, out_ref):
    x = lax.dot_general(
        w_ref[...], embt_ref[...],
        (((1,), (0,)), ((), ())),
        preferred_element_type=jnp.float32,
    )
    x = x + b_ref[...]
    out_ref[...] = lax.pad(x, jnp.float32(0.0),
                           ((0, PAD - NUM_CLASSES, 0), (0, 0, 0)))


def _project(emb_t, w, bcol):
    blk = 16384
    return pl.pallas_call(
        _proj_body,
        grid=((VPAD + blk - 1) // blk,),
        in_specs=[pl.BlockSpec((NUM_CLASSES, HIDDEN), lambda i: (0, 0)),
                  pl.BlockSpec((NUM_CLASSES, 1), lambda i: (0, 0)),
                  pl.BlockSpec((HIDDEN, blk), lambda i: (0, i))],
        out_specs=pl.BlockSpec((PAD, blk), lambda i: (0, i)),
        out_shape=jax.ShapeDtypeStruct((PAD, VPAD), jnp.float32),
    )(w, bcol, emb_t)


def _sc_transpose(pt):
    mesh = plsc.VectorSubcoreMesh(core_axis_name="c", subcore_axis_name="s")
    CPW = VPAD // NW
    NG = CPW // 16

    HALF = CPW // 2
    NGH = HALF // 16

    @functools.partial(
        pl.kernel,
        mesh=mesh,
        out_type=jax.ShapeDtypeStruct((VPAD, PAD), jnp.float32),
        scratch_types=[
            pltpu.VMEM((PAD, HALF), jnp.float32),
            pltpu.VMEM((PAD, HALF), jnp.float32),
            pltpu.VMEM((HALF, PAD), jnp.float32),
            pltpu.VMEM((HALF, PAD), jnp.float32),
            pltpu.SemaphoreType.DMA,
            pltpu.SemaphoreType.DMA,
            pltpu.SemaphoreType.DMA,
            pltpu.SemaphoreType.DMA,
        ],
        compiler_params=pltpu.CompilerParams(use_tc_tiling_on_sc=False,
                                             needs_layout_passes=False),
    )
    def k(pt_hbm, out_hbm, slab0, slab1, out0, out1, si0, si1, so0, so1):
        wid = lax.axis_index("s") * NC + lax.axis_index("c")
        base = wid * CPW
        lanes = lax.iota(jnp.int32, 16)
        slab = (slab0, slab1)
        outv = (out0, out1)

        cpi = [pltpu.async_copy(pt_hbm.at[:, pl.ds(base, HALF)], slab0, si0),
               pltpu.async_copy(pt_hbm.at[:, pl.ds(base + HALF, HALF)],
                                slab1, si1)]
        cpo = [None, None]
        for h in range(2):
            cpi[h].wait()

            def grp(g, carry, _h=h):
                col0 = g * 16
                for r in range(PAD):
                    v = slab[_h][r, pl.ds(col0, 16)]
                    plsc.store_scatter(
                        outv[_h],
                        [col0 + lanes, jnp.full((16,), r, jnp.int32)], v)
                return carry

            lax.fori_loop(0, NGH, grp, 0)
            cpo[h] = pltpu.async_copy(
                outv[h], out_hbm.at[pl.ds(base + h * HALF, HALF)],
                so0 if h == 0 else so1)
        cpo[0].wait()
        cpo[1].wait()

    return k(pt)


def _sc_pool(ids_flat, ptab):
    mesh = plsc.VectorSubcoreMesh(core_axis_name="c", subcore_axis_name="s")

    @functools.partial(
        pl.kernel,
        mesh=mesh,
        out_type=jax.ShapeDtypeStruct((BATCH, PAD), jnp.float32),
        scratch_types=[
            pltpu.VMEM((BPW * SEQ,), jnp.int32),
            pltpu.VMEM((ROWS, PAD), jnp.float32),
            pltpu.VMEM((ROWS, PAD), jnp.float32),
            pltpu.VMEM((BPW, PAD), jnp.float32),
            pltpu.SemaphoreType.DMA,
            pltpu.SemaphoreType.DMA,
        ],
        compiler_params=pltpu.CompilerParams(use_tc_tiling_on_sc=False),
    )
    def k(ids_hbm, ptab_hbm, out_hbm,
          idx_v, rows0, rows1, out_v, sem0, sem1):
        wid = lax.axis_index("s") * NC + lax.axis_index("c")
        base = wid * BPW
        pltpu.sync_copy(ids_hbm.at[pl.ds(base * SEQ, BPW * SEQ)], idx_v)
        inv = jnp.float32(1.0 / SEQ)

        rows = (rows0, rows1)
        sem = (sem0, sem1)

        def start(c, p):
            return pltpu.async_copy(
                ptab_hbm.at[idx_v.at[pl.ds(c * ROWS, ROWS)]], rows[p],
                sem[p])

        def compute(c, p):
            rv = rows[p]

            def row_body(bi, carry):
                r0 = bi * SEQ

                def acc_body(j, acc):
                    o = r0 + j * 8
                    s = ((rv[o] + rv[o + 1]) + (rv[o + 2] + rv[o + 3])) \
                        + ((rv[o + 4] + rv[o + 5]) + (rv[o + 6] + rv[o + 7]))
                    return acc + s

                acc = lax.fori_loop(0, SEQ // 8, acc_body,
                                    jnp.zeros((PAD,), jnp.float32))
                out_v[c * CB + bi] = acc * inv
                return carry

            lax.fori_loop(0, CB, row_body, 0)

        cps = [start(0, 0), start(1, 1)]
        for c in range(NCHUNK):
            p = c % 2
            cps[p].wait()
            compute(c, p)
            if c + 2 < NCHUNK:
                cps[p] = start(c + 2, p)

        pltpu.sync_copy(out_v, out_hbm.at[pl.ds(base, BPW)])

    return k(ids_flat, ptab)


def kernel(input_ids, emb_table, W, b):
    bcol = b.reshape(NUM_CLASSES, 1)
    ptab = _sc_transpose(_project(emb_table.T, W, bcol))
    ids_flat = input_ids.reshape(-1).astype(jnp.int32)
    out = _sc_pool(ids_flat, ptab)
    return out[:, :NUM_CLASSES]


# speedup vs baseline: 1.3599x; 1.0061x over previous
import functools

import jax
import jax.numpy as jnp
from jax import lax
from jax.experimental import pallas as pl
from jax.experimental.pallas import tpu as pltpu
from jax.experimental.pallas import tpu_sc as plsc

VOCAB = 100000
HIDDEN = 64
NUM_CLASSES = 3
BATCH = 4096
SEQ = 200

PAD = 16
NC, NS = 2, 16
NW = NC * NS
VPAD = 100352
BPW = BATCH // NW
CB = 8
NCHUNK = BPW // CB
ROWS = CB * SEQ


def _proj_body(w_ref, b_ref, embt_ref, out_ref):
    x = lax.dot_general(
        w_ref[...], embt_ref[...],
        (((1,), (0,)), ((), ())),
        preferred_element_type=jnp.float32,
    )
    x = x + b_ref[...]
    out_ref[...] = lax.pad(x, jnp.float32(0.0),
                           ((0, PAD - NUM_CLASSES, 0), (0, 0, 0)))


def _project(emb_t, w, bcol):
    blk = 25088
    return pl.pallas_call(
        _proj_body,
        grid=((VPAD + blk - 1) // blk,),
        in_specs=[pl.BlockSpec((NUM_CLASSES, HIDDEN), lambda i: (0, 0)),
                  pl.BlockSpec((NUM_CLASSES, 1), lambda i: (0, 0)),
                  pl.BlockSpec((HIDDEN, blk), lambda i: (0, i))],
        out_specs=pl.BlockSpec((PAD, blk), lambda i: (0, i)),
        out_shape=jax.ShapeDtypeStruct((PAD, VPAD), jnp.float32),
    )(w, bcol, emb_t)


def _sc_transpose(pt):
    mesh = plsc.VectorSubcoreMesh(core_axis_name="c", subcore_axis_name="s")
    CPW = VPAD // NW
    NG = CPW // 16

    HALF = CPW // 2
    NGH = HALF // 16

    @functools.partial(
        pl.kernel,
        mesh=mesh,
        out_type=jax.ShapeDtypeStruct((VPAD, PAD), jnp.float32),
        scratch_types=[
            pltpu.VMEM((PAD, HALF), jnp.float32),
            pltpu.VMEM((PAD, HALF), jnp.float32),
            pltpu.VMEM((HALF, PAD), jnp.float32),
            pltpu.VMEM((HALF, PAD), jnp.float32),
            pltpu.SemaphoreType.DMA,
            pltpu.SemaphoreType.DMA,
            pltpu.SemaphoreType.DMA,
            pltpu.SemaphoreType.DMA,
        ],
        compiler_params=pltpu.CompilerParams(use_tc_tiling_on_sc=False,
                                             needs_layout_passes=False),
    )
    def k(pt_hbm, out_hbm, slab0, slab1, out0, out1, si0, si1, so0, so1):
        wid = lax.axis_index("s") * NC + lax.axis_index("c")
        base = wid * CPW
        lanes = lax.iota(jnp.int32, 16)
        slab = (slab0, slab1)
        outv = (out0, out1)

        cpi = [pltpu.async_copy(pt_hbm.at[:, pl.ds(base, HALF)], slab0, si0),
               pltpu.async_copy(pt_hbm.at[:, pl.ds(base + HALF, HALF)],
                                slab1, si1)]
        cpo = [None, None]
        for h in range(2):
            cpi[h].wait()

            def grp(g, carry, _h=h):
                col0 = g * 16
                for r in range(PAD):
                    v = slab[_h][r, pl.ds(col0, 16)]
                    plsc.store_scatter(
                        outv[_h],
                        [col0 + lanes, jnp.full((16,), r, jnp.int32)], v)
                return carry

            lax.fori_loop(0, NGH, grp, 0)
            cpo[h] = pltpu.async_copy(
                outv[h], out_hbm.at[pl.ds(base + h * HALF, HALF)],
                so0 if h == 0 else so1)
        cpo[0].wait()
        cpo[1].wait()

    return k(pt)


def _sc_pool(ids_flat, ptab):
    mesh = plsc.VectorSubcoreMesh(core_axis_name="c", subcore_axis_name="s")

    @functools.partial(
        pl.kernel,
        mesh=mesh,
        out_type=jax.ShapeDtypeStruct((BATCH, PAD), jnp.float32),
        scratch_types=[
            pltpu.VMEM((BPW * SEQ,), jnp.int32),
            pltpu.VMEM((ROWS, PAD), jnp.float32),
            pltpu.VMEM((ROWS, PAD), jnp.float32),
            pltpu.VMEM((BPW, PAD), jnp.float32),
            pltpu.SemaphoreType.DMA,
            pltpu.SemaphoreType.DMA,
        ],
        compiler_params=pltpu.CompilerParams(use_tc_tiling_on_sc=False),
    )
    def k(ids_hbm, ptab_hbm, out_hbm,
          idx_v, rows0, rows1, out_v, sem0, sem1):
        wid = lax.axis_index("s") * NC + lax.axis_index("c")
        base = wid * BPW
        pltpu.sync_copy(ids_hbm.at[pl.ds(base * SEQ, BPW * SEQ)], idx_v)
        inv = jnp.float32(1.0 / SEQ)

        rows = (rows0, rows1)
        sem = (sem0, sem1)

        def start(c, p):
            return pltpu.async_copy(
                ptab_hbm.at[idx_v.at[pl.ds(c * ROWS, ROWS)]], rows[p],
                sem[p])

        def compute(c, p):
            rv = rows[p]

            def row_body(bi, carry):
                r0 = bi * SEQ

                def acc_body(j, acc):
                    o = r0 + j * 8
                    s = ((rv[o] + rv[o + 1]) + (rv[o + 2] + rv[o + 3])) \
                        + ((rv[o + 4] + rv[o + 5]) + (rv[o + 6] + rv[o + 7]))
                    return acc + s

                acc = lax.fori_loop(0, SEQ // 8, acc_body,
                                    jnp.zeros((PAD,), jnp.float32))
                out_v[c * CB + bi] = acc * inv
                return carry

            lax.fori_loop(0, CB, row_body, 0)

        cps = [start(0, 0), start(1, 1)]
        for c in range(NCHUNK):
            p = c % 2
            cps[p].wait()
            compute(c, p)
            if c + 2 < NCHUNK:
                cps[p] = start(c + 2, p)

        pltpu.sync_copy(out_v, out_hbm.at[pl.ds(base, BPW)])

    return k(ids_flat, ptab)


def kernel(input_ids, emb_table, W, b):
    bcol = b.reshape(NUM_CLASSES, 1)
    ptab = _sc_transpose(_project(emb_table.T, W, bcol))
    ids_flat = input_ids.reshape(-1).astype(jnp.int32)
    out = _sc_pool(ids_flat, ptab)
    return out[:, :NUM_CLASSES]
